# baseline (speedup 1.0000x reference)
import functools

import jax
import jax.numpy as jnp
from jax import lax
from jax.experimental import pallas as pl
from jax.experimental.pallas import tpu as pltpu
from jax.experimental.pallas import tpu_sc as plsc

CHUNK = 128
NBUF = 8
LOOKAHEAD = 6
NW = 32
LANES = 16


def _sc_embed(idx, token_table, pos2):
    (nrow,) = idx.shape
    vocab, d = token_table.shape
    s2 = pos2.shape[0]
    s = s2 // 2
    rpw = nrow // NW
    cpw = rpw // CHUNK
    assert rpw % CHUNK == 0 and cpw % NBUF == 0 and rpw % s == 0
    assert d % LANES == 0 and NBUF > LOOKAHEAD >= 2

    mesh = plsc.VectorSubcoreMesh(core_axis_name="c", subcore_axis_name="s")

    @functools.partial(
        pl.kernel,
        mesh=mesh,
        out_type=jax.ShapeDtypeStruct((nrow, d), jnp.float32),
        compiler_params=pltpu.CompilerParams(use_tc_tiling_on_sc=False),
        scratch_types=[
            pltpu.VMEM((s2, d), jnp.float32),
            pltpu.VMEM((rpw,), jnp.int32),
        ]
        + [pltpu.VMEM((CHUNK, d), jnp.float32)] * NBUF
        + [pltpu.SemaphoreType.DMA] * (2 * NBUF),
    )
    def body(idx_hbm, tab_hbm, pos_hbm, out_hbm, pos_v, idx_v, *rest):
        rows = rest[:NBUF]
        sems = rest[NBUF:]
        gsem = sems[:NBUF]
        ssem = sems[NBUF:]
        wid = lax.axis_index("s") * 2 + lax.axis_index("c")
        base = wid * rpw
        pltpu.sync_copy(pos_hbm, pos_v)
        pltpu.sync_copy(idx_hbm.at[pl.ds(base, rpw)], idx_v)

        def fire_gather(c, b):
            pltpu.async_copy(
                tab_hbm.at[idx_v.at[pl.ds(c * CHUNK, CHUNK)]],
                rows[b],
                gsem[b],
            )

        def wait_chunk(sem, b):
            pltpu.make_async_copy(
                out_hbm.at[pl.ds(0, CHUNK)], rows[b], sem
            ).wait()

        for c0 in range(LOOKAHEAD):
            fire_gather(c0, c0)

        def phase(i, b):
            c = i * NBUF + b
            wait_chunk(gsem[b], b)
            p0 = lax.rem(c * CHUNK, s)

            def radd(r, rc):
                for v in range(d // LANES):
                    vs = pl.ds(v * LANES, LANES)
                    rows[b][r, vs] = rows[b][r, vs] + pos_v[p0 + r, vs]
                return rc

            lax.fori_loop(0, CHUNK, radd, 0)
            pltpu.async_copy(
                rows[b], out_hbm.at[pl.ds(base + c * CHUNK, CHUNK)], ssem[b]
            )

            c2 = c + LOOKAHEAD
            b2 = (b + LOOKAHEAD) % NBUF

            @pl.when(c2 < cpw)
            def _():
                @pl.when(c2 >= NBUF)
                def _():
                    wait_chunk(ssem[b2], b2)

                fire_gather(c2, b2)

            return 0

        def blk_cycle(i, carry):
            for b in range(NBUF):
                phase(i, b)
            return carry

        lax.fori_loop(0, cpw // NBUF, blk_cycle, 0)
        for b in range(NBUF):
            wait_chunk(ssem[b], b)

    return body(idx, token_table, pos2)


def kernel(inputs, token_table, pos_table):
    nb, s = inputs.shape
    _, d = token_table.shape
    idx_flat = inputs.astype(jnp.int32).reshape(-1)
    pos2 = jnp.concatenate([pos_table, pos_table], axis=0)
    out = _sc_embed(idx_flat, token_table, pos2)
    return out.reshape(nb, s, d)


# speedup vs baseline: 1.3153x; 1.3153x over previous
import functools

import jax
import jax.numpy as jnp
from jax import lax
from jax.experimental import pallas as pl
from jax.experimental.pallas import tpu as pltpu
from jax.experimental.pallas import tpu_sc as plsc

GSIZES = (128, 72)
UNIT = 2
NBUF = 3
LOOKAHEAD = 2
NW = 32
LANES = 16


def _sc_embed(idx, token_table, pos_table):
    nseq, s = idx.shape
    vocab, d = token_table.shape
    spw = nseq // NW
    upw = spw // UNIT
    assert s == sum(GSIZES) and nseq % NW == 0 and spw % UNIT == 0
    assert d % LANES == 0 and NBUF > LOOKAHEAD

    mesh = plsc.VectorSubcoreMesh(core_axis_name="c", subcore_axis_name="s")

    @functools.partial(
        pl.kernel,
        mesh=mesh,
        out_type=jax.ShapeDtypeStruct((nseq, s, d), jnp.float32),
        compiler_params=pltpu.CompilerParams(use_tc_tiling_on_sc=False),
        scratch_types=[
            pltpu.VMEM((s, d), jnp.float32),
            pltpu.VMEM((spw, s), jnp.int32),
        ]
        + [pltpu.VMEM((UNIT, s, d), jnp.float32)] * NBUF
        + [pltpu.SemaphoreType.DMA] * (2 * NBUF),
    )
    def body(idx_hbm, tab_hbm, pos_hbm, out_hbm, pos_v, idx_v, *rest):
        rows = rest[:NBUF]
        sems = rest[NBUF:]
        gsem = sems[:NBUF]
        ssem = sems[NBUF:]
        wid = lax.axis_index("s") * 2 + lax.axis_index("c")
        base = wid * spw
        pltpu.sync_copy(pos_hbm, pos_v)
        pltpu.sync_copy(idx_hbm.at[pl.ds(base, spw)], idx_v)

        def fire_gathers(u, b):
            for j in range(UNIT):
                off = 0
                for n in GSIZES:
                    pltpu.async_copy(
                        tab_hbm.at[idx_v.at[u * UNIT + j, pl.ds(off, n)]],
                        rows[b].at[j, pl.ds(off, n)],
                        gsem[b],
                    )
                    off += n

        def wait_unit(sem, b):
            pltpu.make_async_copy(
                out_hbm.at[pl.ds(0, UNIT)], rows[b], sem
            ).wait()

        for u0 in range(LOOKAHEAD):
            fire_gathers(u0, u0)

        def phase(i, b):
            u = i * NBUF + b
            wait_unit(gsem[b], b)

            def radd(r, rc):
                for j in range(UNIT):
                    for v in range(d // LANES):
                        vs = pl.ds(v * LANES, LANES)
                        rows[b][j, r, vs] = rows[b][j, r, vs] + pos_v[r, vs]
                return rc

            lax.fori_loop(0, s, radd, 0)
            pltpu.async_copy(
                rows[b], out_hbm.at[pl.ds(base + u * UNIT, UNIT)], ssem[b]
            )

            u2 = u + LOOKAHEAD
            b2 = (b + LOOKAHEAD) % NBUF

            @pl.when(u2 < upw)
            def _():
                @pl.when(u2 >= NBUF)
                def _():
                    wait_unit(ssem[b2], b2)

                fire_gathers(u2, b2)

            return 0

        def blk_cycle(i, carry):
            for b in range(NBUF):
                phase(i, b)
            return carry

        lax.fori_loop(0, upw // NBUF, blk_cycle, 0)
        for r in range(upw % NBUF):
            phase(jnp.int32(upw // NBUF), r)
        for b in range(NBUF):
            wait_unit(ssem[b], b)

    return body(idx, token_table, pos_table)


def kernel(inputs, token_table, pos_table):
    return _sc_embed(inputs.astype(jnp.int32), token_table, pos_table)
